# Initial kernel scaffold; baseline (speedup 1.0000x reference)
import jax
import jax.numpy as jnp
from jax.experimental import pallas as pl


def kernel(x):
    raise NotImplementedError("write your pallas kernel here")


# speedup vs baseline: 180.3254x; 180.3254x over previous
import functools

import jax
import jax.numpy as jnp
from jax import lax
from jax.experimental import pallas as pl
from jax.experimental.pallas import tpu as pltpu
from jax.experimental.pallas import tpu_sc as plsc

PRE = 3000
NCLS = 8
SLOTS = 3008
NCHUNK = SLOTS // 16
POST = 1000
THRESH = 0.5
OUT_ROWS = 2080
PER_W2 = 752
BIG = SLOTS

_mesh = plsc.VectorSubcoreMesh(core_axis_name="c", subcore_axis_name="s")
_cparams = pltpu.CompilerParams(needs_layout_passes=False,
                                use_tc_tiling_on_sc=False)


def _wid():
    return lax.axis_index("s") * 2 + lax.axis_index("c")


@functools.partial(
    pl.kernel,
    mesh=_mesh,
    out_type=[
        jax.ShapeDtypeStruct((NCLS, SLOTS, 16), jnp.float32),
        jax.ShapeDtypeStruct((NCLS * SLOTS,), jnp.float32),
        jax.ShapeDtypeStruct((NCLS * 16,), jnp.int32),
    ],
    scratch_types=[
        pltpu.VMEM((PRE * 6,), jnp.float32),
        pltpu.VMEM((SLOTS, 16), jnp.float32),
        pltpu.VMEM((SLOTS,), jnp.float32),
        pltpu.VMEM((16,), jnp.int32),
        pltpu.SMEM((1,), jnp.int32),
    ],
    compiler_params=_cparams,
)
def _wbf_cluster(x_hbm, rows_hbm, keys_hbm, counts_hbm, x_v, state_v,
                 keys_v, m_v, m_smem):
    wid = _wid()

    @pl.when(wid < NCLS)
    def _():
        c = wid
        cf = c.astype(jnp.float32)
        pltpu.sync_copy(x_hbm, x_v)
        m_smem[0] = jnp.int32(0)
        iota = lax.iota(jnp.int32, 16)
        lanes = iota
        zeros16i = jnp.zeros((16,), jnp.int32)

        def step(j, carry):
            base = j * 6
            xrow = plsc.load_gather(x_v, [base + jnp.minimum(iota, 5)])

            @pl.when(xrow[5] == cf)
            def _():
                bx1 = xrow[0]
                by1 = xrow[1]
                bx2 = xrow[2]
                by2 = xrow[3]
                sc = xrow[4]
                barea = (bx2 - bx1) * (by2 - by1)
                m = m_smem[0]
                nch = (m + 15) // 16
                vbx1 = jnp.full((16,), bx1)
                vby1 = jnp.full((16,), by1)
                vbx2 = jnp.full((16,), bx2)
                vby2 = jnp.full((16,), by2)
                vba = jnp.full((16,), barea)
                vthr = jnp.float32(THRESH)

                def chunk(t, minvec):
                    ridx = t * 16 + iota
                    mx1 = plsc.load_gather(state_v, [ridx, zeros16i])
                    my1 = plsc.load_gather(state_v, [ridx, zeros16i + 1])
                    mx2 = plsc.load_gather(state_v, [ridx, zeros16i + 2])
                    my2 = plsc.load_gather(state_v, [ridx, zeros16i + 3])
                    ltx = jnp.maximum(mx1, vbx1)
                    lty = jnp.maximum(my1, vby1)
                    rbx = jnp.minimum(mx2, vbx2)
                    rby = jnp.minimum(my2, vby2)
                    iw = jnp.maximum(rbx - ltx, 0.0)
                    ih = jnp.maximum(rby - lty, 0.0)
                    inter = iw * ih
                    mab = (mx2 - mx1) * (my2 - my1)
                    union = vba + mab - inter
                    hits = (inter > vthr * union) & (ridx < m)
                    return jnp.minimum(minvec, jnp.where(hits, ridx, BIG))

                minvec = lax.fori_loop(0, nch, chunk, jnp.full((16,), BIG))
                hitpos = jnp.min(minvec)
                hit = hitpos < m
                idx = jnp.where(hit, hitpos, m)

                oldrow = state_v[idx, :]
                w1 = oldrow[6] + sc * bx1
                w2 = oldrow[7] + sc * by1
                w3 = oldrow[8] + sc * bx2
                w4 = oldrow[9] + sc * by2
                sn = oldrow[10] + sc
                cn = oldrow[11] + 1.0
                numer = jnp.where(
                    lanes == 0, w1,
                    jnp.where(lanes == 1, w2,
                              jnp.where(lanes == 2, w3,
                                        jnp.where(lanes == 3, w4, sn))))
                denom = jnp.where(lanes < 4, sn,
                                  jnp.where(lanes == 4, cn, 1.0))
                q = numer / denom
                hitrow = jnp.where(
                    lanes <= 4, q,
                    jnp.where(lanes == 5, cf,
                              jnp.where(lanes == 6, w1,
                                        jnp.where(lanes == 7, w2,
                                                  jnp.where(lanes == 8, w3,
                                                            jnp.where(lanes == 9, w4,
                                                                      jnp.where(lanes == 10, sn, cn)))))))
                crow = jnp.where(
                    lanes == 0, bx1,
                    jnp.where(lanes == 1, by1,
                              jnp.where(lanes == 2, bx2,
                                        jnp.where(lanes == 3, by2,
                                                  jnp.where(lanes == 4, sc,
                                                            jnp.where(lanes == 5, cf,
                                                                      jnp.where(lanes == 6, sc * bx1,
                                                                                jnp.where(lanes == 7, sc * by1,
                                                                                          jnp.where(lanes == 8, sc * bx2,
                                                                                                    jnp.where(lanes == 9, sc * by2,
                                                                                                              jnp.where(lanes == 10, sc, 1.0)))))))))))
                state_v[idx, :] = jnp.where(hit, hitrow, crow)
                m_smem[0] = jnp.where(hit, m, m + 1)

            return carry

        lax.fori_loop(0, PRE, step, 0)

        m = m_smem[0]

        def key_chunk(t, carry):
            ridx = t * 16 + iota
            scores = plsc.load_gather(state_v, [ridx, zeros16i + 4])
            keys_v[pl.ds(t * 16, 16)] = jnp.where(ridx < m, scores, -1.0)
            return carry

        lax.fori_loop(0, NCHUNK, key_chunk, 0)

        m_v[...] = jnp.full((16,), m, jnp.int32)
        pltpu.sync_copy(state_v, rows_hbm.at[c])
        pltpu.sync_copy(keys_v, keys_hbm.at[pl.ds(c * SLOTS, SLOTS)])
        pltpu.sync_copy(m_v, counts_hbm.at[pl.ds(c * 16, 16)])


@functools.partial(
    pl.kernel,
    mesh=_mesh,
    out_type=jax.ShapeDtypeStruct((OUT_ROWS, 16), jnp.float32),
    scratch_types=[
        pltpu.VMEM((NCLS * SLOTS,), jnp.float32),
        pltpu.VMEM((NCLS * 16,), jnp.int32),
        pltpu.VMEM((PER_W2 + 16, 16), jnp.float32),
        pltpu.VMEM((6, 128), jnp.int32),
        pltpu.VMEM((1024, 16), jnp.float32),
        pltpu.SemaphoreType.DMA,
    ],
    compiler_params=_cparams,
)
def _wbf_topk(rows_hbm, keys_hbm, counts_hbm, out_hbm, keys_v, counts_v,
              rows_v, idx_v, zbuf, sem):
    wid = _wid()
    c = wid // 4
    s0 = (wid % 4) * PER_W2
    pltpu.sync_copy(keys_hbm, keys_v)
    pltpu.sync_copy(counts_hbm, counts_v)
    pltpu.sync_copy(rows_hbm.at[c, pl.ds(s0, PER_W2)],
                    rows_v.at[pl.ds(0, PER_W2)])
    iota = lax.iota(jnp.int32, 16)
    lanes = iota
    dummy = jnp.int32(1000) + wid

    def _rank(key, g):
        keyv = jnp.full((16,), key)
        gv = jnp.full((16,), g)

        def cls_loop(cc, acc):
            mcc = counts_v[pl.ds(cc * 16, 16)][0]
            nch = (mcc + 15) // 16
            base = cc * SLOTS

            def ch(t, acc):
                v = keys_v[pl.ds(base + t * 16, 16)]
                lidx = base + t * 16 + iota
                lvalid = (t * 16 + iota) < mcc
                msk = ((v > keyv) | ((v == keyv) & (lidx < gv))) & lvalid
                return acc + plsc.all_reduce_population_count(msk)

            return lax.fori_loop(0, nch, ch, acc)

        acc = lax.fori_loop(0, NCLS, cls_loop, jnp.zeros((16,), jnp.int32))
        rank = jnp.max(acc)
        return jnp.where(rank < POST, rank, dummy)

    def group(t, carry):
        def one(k, dv):
            g = c * SLOTS + s0 + t * 16 + k
            key = plsc.load_gather(keys_v, [jnp.full((16,), g)])[0]
            dest = lax.cond(key >= 0.0, lambda: _rank(key, g),
                            lambda: dummy)
            return jnp.where(lanes == k, dest, dv)

        dv = lax.fori_loop(0, 16, one, jnp.full((16,), dummy))
        idx_v[t // 8, pl.ds((t % 8) * 16, 16)] = dv
        return carry

    lax.fori_loop(0, PER_W2 // 16, group, 0)
    idx_v[5, pl.ds(112, 16)] = jnp.full((16,), dummy)

    copies = []
    for j in range(6):
        copies.append(
            pltpu.async_copy(rows_v.at[pl.ds(j * 128, 128)],
                             out_hbm.at[idx_v.at[j]], sem))
    for cp in copies:
        cp.wait()

    @pl.when(wid == 0)
    def _():
        def zf(t, carry):
            zbuf[t, :] = jnp.zeros((16,), jnp.float32)
            return carry

        lax.fori_loop(0, 1024, zf, 0)
        mtot = jnp.int32(0)
        for cc in range(NCLS):
            mtot = mtot + counts_v[pl.ds(cc * 16, 16)][0]
        zstart = jnp.minimum(mtot, jnp.int32(POST))
        pltpu.sync_copy(zbuf, out_hbm.at[pl.ds(zstart, 1024)])


def kernel(x):
    x3k = jnp.reshape(x[:PRE].astype(jnp.float32), (-1,))
    rows, keys, counts = _wbf_cluster(x3k)
    out = _wbf_topk(rows, keys, counts)
    return out[:POST, :6]


# speedup vs baseline: 323.1611x; 1.7921x over previous
import functools

import jax
import jax.numpy as jnp
from jax import lax
from jax.experimental import pallas as pl
from jax.experimental.pallas import tpu as pltpu
from jax.experimental.pallas import tpu_sc as plsc

PRE = 3000
NCLS = 8
SLOTS = 3008
NCHUNK = SLOTS // 16
POST = 1000
THRESH = 0.5
OUT_ROWS = 2080
PER_W2 = 752
BIG = SLOTS

_mesh = plsc.VectorSubcoreMesh(core_axis_name="c", subcore_axis_name="s")
_cparams = pltpu.CompilerParams(needs_layout_passes=False,
                                use_tc_tiling_on_sc=False)


def _wid():
    return lax.axis_index("s") * 2 + lax.axis_index("c")


@functools.partial(
    pl.kernel,
    mesh=_mesh,
    out_type=[
        jax.ShapeDtypeStruct((NCLS, SLOTS, 16), jnp.float32),
        jax.ShapeDtypeStruct((NCLS * SLOTS,), jnp.float32),
        jax.ShapeDtypeStruct((NCLS * 16,), jnp.int32),
    ],
    scratch_types=[
        pltpu.VMEM((PRE * 6,), jnp.float32),
        pltpu.VMEM((SLOTS, 16), jnp.float32),
        pltpu.VMEM((SLOTS,), jnp.float32),
        pltpu.VMEM((16,), jnp.int32),
        pltpu.SMEM((1,), jnp.int32),
    ],
    compiler_params=_cparams,
)
def _wbf_cluster(x_hbm, rows_hbm, keys_hbm, counts_hbm, x_v, state_v,
                 keys_v, m_v, m_smem):
    wid = _wid()

    @pl.when(wid < NCLS)
    def _():
        c = wid
        cf = c.astype(jnp.float32)
        pltpu.sync_copy(x_hbm, x_v)
        m_smem[0] = jnp.int32(0)
        iota = lax.iota(jnp.int32, 16)
        lanes = iota
        zeros16i = jnp.zeros((16,), jnp.int32)

        def step(j, carry):
            base = j * 6
            xrow = plsc.load_gather(x_v, [base + jnp.minimum(iota, 5)])

            @pl.when(xrow[5] == cf)
            def _():
                bx1 = xrow[0]
                by1 = xrow[1]
                bx2 = xrow[2]
                by2 = xrow[3]
                sc = xrow[4]
                barea = (bx2 - bx1) * (by2 - by1)
                m = m_smem[0]
                nch = (m + 15) // 16
                vbx1 = jnp.full((16,), bx1)
                vby1 = jnp.full((16,), by1)
                vbx2 = jnp.full((16,), bx2)
                vby2 = jnp.full((16,), by2)
                vba = jnp.full((16,), barea)
                vthr = jnp.float32(THRESH)

                def chunk(t, minvec):
                    ridx = t * 16 + iota
                    mx1 = plsc.load_gather(state_v, [ridx, zeros16i])
                    my1 = plsc.load_gather(state_v, [ridx, zeros16i + 1])
                    mx2 = plsc.load_gather(state_v, [ridx, zeros16i + 2])
                    my2 = plsc.load_gather(state_v, [ridx, zeros16i + 3])
                    ltx = jnp.maximum(mx1, vbx1)
                    lty = jnp.maximum(my1, vby1)
                    rbx = jnp.minimum(mx2, vbx2)
                    rby = jnp.minimum(my2, vby2)
                    iw = jnp.maximum(rbx - ltx, 0.0)
                    ih = jnp.maximum(rby - lty, 0.0)
                    inter = iw * ih
                    mab = (mx2 - mx1) * (my2 - my1)
                    union = vba + mab - inter
                    hits = (inter > vthr * union) & (ridx < m)
                    return jnp.minimum(minvec, jnp.where(hits, ridx, BIG))

                minvec = lax.fori_loop(0, nch, chunk, jnp.full((16,), BIG))
                hitpos = jnp.min(minvec)
                hit = hitpos < m
                idx = jnp.where(hit, hitpos, m)

                oldrow = state_v[idx, :]
                w1 = oldrow[6] + sc * bx1
                w2 = oldrow[7] + sc * by1
                w3 = oldrow[8] + sc * bx2
                w4 = oldrow[9] + sc * by2
                sn = oldrow[10] + sc
                cn = oldrow[11] + 1.0
                numer = jnp.where(
                    lanes == 0, w1,
                    jnp.where(lanes == 1, w2,
                              jnp.where(lanes == 2, w3,
                                        jnp.where(lanes == 3, w4, sn))))
                denom = jnp.where(lanes < 4, sn,
                                  jnp.where(lanes == 4, cn, 1.0))
                q = numer / denom
                hitrow = jnp.where(
                    lanes <= 4, q,
                    jnp.where(lanes == 5, cf,
                              jnp.where(lanes == 6, w1,
                                        jnp.where(lanes == 7, w2,
                                                  jnp.where(lanes == 8, w3,
                                                            jnp.where(lanes == 9, w4,
                                                                      jnp.where(lanes == 10, sn, cn)))))))
                crow = jnp.where(
                    lanes == 0, bx1,
                    jnp.where(lanes == 1, by1,
                              jnp.where(lanes == 2, bx2,
                                        jnp.where(lanes == 3, by2,
                                                  jnp.where(lanes == 4, sc,
                                                            jnp.where(lanes == 5, cf,
                                                                      jnp.where(lanes == 6, sc * bx1,
                                                                                jnp.where(lanes == 7, sc * by1,
                                                                                          jnp.where(lanes == 8, sc * bx2,
                                                                                                    jnp.where(lanes == 9, sc * by2,
                                                                                                              jnp.where(lanes == 10, sc, 1.0)))))))))))
                state_v[idx, :] = jnp.where(hit, hitrow, crow)
                m_smem[0] = jnp.where(hit, m, m + 1)

            return carry

        lax.fori_loop(0, PRE, step, 0)

        m = m_smem[0]

        def key_chunk(t, carry):
            ridx = t * 16 + iota
            scores = plsc.load_gather(state_v, [ridx, zeros16i + 4])
            keys_v[pl.ds(t * 16, 16)] = jnp.where(ridx < m, scores, -1.0)
            return carry

        lax.fori_loop(0, NCHUNK, key_chunk, 0)

        m_v[...] = jnp.full((16,), m, jnp.int32)
        pltpu.sync_copy(state_v, rows_hbm.at[c])
        pltpu.sync_copy(keys_v, keys_hbm.at[pl.ds(c * SLOTS, SLOTS)])
        pltpu.sync_copy(m_v, counts_hbm.at[pl.ds(c * 16, 16)])


@functools.partial(
    pl.kernel,
    mesh=_mesh,
    out_type=jax.ShapeDtypeStruct((OUT_ROWS, 16), jnp.float32),
    scratch_types=[
        pltpu.VMEM((NCLS * SLOTS,), jnp.float32),
        pltpu.VMEM((NCLS * 16,), jnp.int32),
        pltpu.VMEM((PER_W2 + 16, 16), jnp.float32),
        pltpu.VMEM((6, 128), jnp.int32),
        pltpu.VMEM((6, 128), jnp.int32),
        pltpu.VMEM((1024, 16), jnp.float32),
        pltpu.SemaphoreType.DMA,
    ],
    compiler_params=_cparams,
)
def _wbf_topk(rows_hbm, keys_hbm, counts_hbm, out_hbm, keys_v, counts_v,
              rows_v, idx_v, gidx_v, zbuf, sem):
    wid = _wid()
    c = wid // 4
    qt = wid % 4
    iota = lax.iota(jnp.int32, 16)
    pltpu.sync_copy(keys_hbm, keys_v)
    pltpu.sync_copy(counts_hbm, counts_v)
    for t in range(PER_W2 // 16):
        vals = (qt + 4 * t) * 16 + iota
        gidx_v[t // 8, pl.ds((t % 8) * 16, 16)] = vals
    gidx_v[5, pl.ds(112, 16)] = jnp.zeros((16,), jnp.int32)
    gcopies = []
    for j in range(6):
        gcopies.append(
            pltpu.async_copy(rows_hbm.at[c].at[gidx_v.at[j]],
                             rows_v.at[pl.ds(j * 128, 128)], sem))
    for cp in gcopies:
        cp.wait()
    lanes = iota
    dummy = jnp.int32(1000) + wid

    def _rank(key, g):
        keyv = jnp.full((16,), key)
        gv = jnp.full((16,), g)

        def cls_loop(cc, acc):
            mcc = counts_v[pl.ds(cc * 16, 16)][0]
            nch = (mcc + 15) // 16
            base = cc * SLOTS

            def ch(t, acc):
                v = keys_v[pl.ds(base + t * 16, 16)]
                lidx = base + t * 16 + iota
                lvalid = (t * 16 + iota) < mcc
                msk = ((v > keyv) | ((v == keyv) & (lidx < gv))) & lvalid
                return acc + plsc.all_reduce_population_count(msk)

            return lax.fori_loop(0, nch, ch, acc)

        acc = lax.fori_loop(0, NCLS, cls_loop, jnp.zeros((16,), jnp.int32))
        rank = jnp.max(acc)
        return jnp.where(rank < POST, rank, dummy)

    def group(t, carry):
        def one(k, dv):
            g = c * SLOTS + (qt + 4 * t) * 16 + k
            key = plsc.load_gather(keys_v, [jnp.full((16,), g)])[0]
            dest = lax.cond(key >= 0.0, lambda: _rank(key, g),
                            lambda: dummy)
            return jnp.where(lanes == k, dest, dv)

        dv = lax.fori_loop(0, 16, one, jnp.full((16,), dummy))
        idx_v[t // 8, pl.ds((t % 8) * 16, 16)] = dv
        return carry

    lax.fori_loop(0, PER_W2 // 16, group, 0)
    idx_v[5, pl.ds(112, 16)] = jnp.full((16,), dummy)

    copies = []
    for j in range(6):
        copies.append(
            pltpu.async_copy(rows_v.at[pl.ds(j * 128, 128)],
                             out_hbm.at[idx_v.at[j]], sem))
    for cp in copies:
        cp.wait()

    @pl.when(wid == 0)
    def _():
        def zf(t, carry):
            zbuf[t, :] = jnp.zeros((16,), jnp.float32)
            return carry

        lax.fori_loop(0, 1024, zf, 0)
        mtot = jnp.int32(0)
        for cc in range(NCLS):
            mtot = mtot + counts_v[pl.ds(cc * 16, 16)][0]
        zstart = jnp.minimum(mtot, jnp.int32(POST))
        pltpu.sync_copy(zbuf, out_hbm.at[pl.ds(zstart, 1024)])


def kernel(x):
    x3k = jnp.reshape(x[:PRE].astype(jnp.float32), (-1,))
    rows, keys, counts = _wbf_cluster(x3k)
    out = _wbf_topk(rows, keys, counts)
    return out[:POST, :6]


# speedup vs baseline: 465.9911x; 1.4420x over previous
import functools

import jax
import jax.numpy as jnp
from jax import lax
from jax.experimental import pallas as pl
from jax.experimental.pallas import tpu as pltpu
from jax.experimental.pallas import tpu_sc as plsc

PRE = 3000
NCLS = 8
SLOTS = 3008
NCHUNK = SLOTS // 16
POST = 1000
THRESH = 0.5
OUT_ROWS = 2080
PER_W2 = 752
BIG = SLOTS

_mesh = plsc.VectorSubcoreMesh(core_axis_name="c", subcore_axis_name="s")
_cparams = pltpu.CompilerParams(needs_layout_passes=False,
                                use_tc_tiling_on_sc=False)


def _wid():
    return lax.axis_index("s") * 2 + lax.axis_index("c")


@functools.partial(
    pl.kernel,
    mesh=_mesh,
    out_type=[
        jax.ShapeDtypeStruct((NCLS, SLOTS, 16), jnp.float32),
        jax.ShapeDtypeStruct((NCLS * SLOTS,), jnp.float32),
        jax.ShapeDtypeStruct((NCLS * 16,), jnp.int32),
    ],
    scratch_types=[
        pltpu.VMEM((PRE * 6,), jnp.float32),
        pltpu.VMEM((SLOTS, 16), jnp.float32),
        pltpu.VMEM((SLOTS,), jnp.float32),
        pltpu.VMEM((16,), jnp.int32),
        pltpu.VMEM((SLOTS,), jnp.int32),
        pltpu.SMEM((2,), jnp.int32),
    ],
    compiler_params=_cparams,
)
def _wbf_cluster(x_hbm, rows_hbm, keys_hbm, counts_hbm, x_v, state_v,
                 keys_v, m_v, act_v, m_smem):
    wid = _wid()

    @pl.when(wid < NCLS)
    def _():
        c = wid
        cf = c.astype(jnp.float32)
        pltpu.sync_copy(x_hbm, x_v)
        m_smem[0] = jnp.int32(0)
        m_smem[1] = jnp.int32(0)
        iota = lax.iota(jnp.int32, 16)
        lanes = iota
        zeros16i = jnp.zeros((16,), jnp.int32)

        def compact(t, carry):
            rows16 = t * 16 + iota
            lblv = plsc.load_gather(
                x_v, [jnp.minimum(rows16 * 6 + 5, PRE * 6 - 1)])
            msk = (lblv == cf) & (rows16 < PRE)
            cur = m_smem[1]
            pos = cur + plsc.cumsum(msk.astype(jnp.int32)) - 1
            plsc.store_scatter(act_v, [pos], rows16, mask=msk)
            m_smem[1] = cur + jnp.max(
                plsc.all_reduce_population_count(msk))
            return carry

        lax.fori_loop(0, NCHUNK, compact, 0)

        def step(i, carry):
            j = plsc.load_gather(act_v, [jnp.full((16,), i)])[0]
            base = j * 6
            xrow = plsc.load_gather(x_v, [base + jnp.minimum(iota, 5)])

            if True:
                bx1 = xrow[0]
                by1 = xrow[1]
                bx2 = xrow[2]
                by2 = xrow[3]
                sc = xrow[4]
                barea = (bx2 - bx1) * (by2 - by1)
                m = m_smem[0]
                nch = (m + 15) // 16
                vbx1 = jnp.full((16,), bx1)
                vby1 = jnp.full((16,), by1)
                vbx2 = jnp.full((16,), bx2)
                vby2 = jnp.full((16,), by2)
                vba = jnp.full((16,), barea)
                vthr = jnp.float32(THRESH)

                def chunk(t, minvec):
                    ridx = t * 16 + iota
                    mx1 = plsc.load_gather(state_v, [ridx, zeros16i])
                    my1 = plsc.load_gather(state_v, [ridx, zeros16i + 1])
                    mx2 = plsc.load_gather(state_v, [ridx, zeros16i + 2])
                    my2 = plsc.load_gather(state_v, [ridx, zeros16i + 3])
                    ltx = jnp.maximum(mx1, vbx1)
                    lty = jnp.maximum(my1, vby1)
                    rbx = jnp.minimum(mx2, vbx2)
                    rby = jnp.minimum(my2, vby2)
                    iw = jnp.maximum(rbx - ltx, 0.0)
                    ih = jnp.maximum(rby - lty, 0.0)
                    inter = iw * ih
                    mab = (mx2 - mx1) * (my2 - my1)
                    union = vba + mab - inter
                    hits = (inter > vthr * union) & (ridx < m)
                    return jnp.minimum(minvec, jnp.where(hits, ridx, BIG))

                minvec = lax.fori_loop(0, nch, chunk, jnp.full((16,), BIG))
                hitpos = jnp.min(minvec)
                hit = hitpos < m
                idx = jnp.where(hit, hitpos, m)

                oldrow = state_v[idx, :]
                w1 = oldrow[6] + sc * bx1
                w2 = oldrow[7] + sc * by1
                w3 = oldrow[8] + sc * bx2
                w4 = oldrow[9] + sc * by2
                sn = oldrow[10] + sc
                cn = oldrow[11] + 1.0
                numer = jnp.where(
                    lanes == 0, w1,
                    jnp.where(lanes == 1, w2,
                              jnp.where(lanes == 2, w3,
                                        jnp.where(lanes == 3, w4, sn))))
                denom = jnp.where(lanes < 4, sn,
                                  jnp.where(lanes == 4, cn, 1.0))
                q = numer / denom
                hitrow = jnp.where(
                    lanes <= 4, q,
                    jnp.where(lanes == 5, cf,
                              jnp.where(lanes == 6, w1,
                                        jnp.where(lanes == 7, w2,
                                                  jnp.where(lanes == 8, w3,
                                                            jnp.where(lanes == 9, w4,
                                                                      jnp.where(lanes == 10, sn, cn)))))))
                crow = jnp.where(
                    lanes == 0, bx1,
                    jnp.where(lanes == 1, by1,
                              jnp.where(lanes == 2, bx2,
                                        jnp.where(lanes == 3, by2,
                                                  jnp.where(lanes == 4, sc,
                                                            jnp.where(lanes == 5, cf,
                                                                      jnp.where(lanes == 6, sc * bx1,
                                                                                jnp.where(lanes == 7, sc * by1,
                                                                                          jnp.where(lanes == 8, sc * bx2,
                                                                                                    jnp.where(lanes == 9, sc * by2,
                                                                                                              jnp.where(lanes == 10, sc, 1.0)))))))))))
                state_v[idx, :] = jnp.where(hit, hitrow, crow)
                m_smem[0] = jnp.where(hit, m, m + 1)

            return carry

        lax.fori_loop(0, m_smem[1], step, 0)

        m = m_smem[0]

        def key_chunk(t, carry):
            ridx = t * 16 + iota
            scores = plsc.load_gather(state_v, [ridx, zeros16i + 4])
            keys_v[pl.ds(t * 16, 16)] = jnp.where(ridx < m, scores, -1.0)
            return carry

        lax.fori_loop(0, NCHUNK, key_chunk, 0)

        m_v[...] = jnp.full((16,), m, jnp.int32)
        pltpu.sync_copy(state_v, rows_hbm.at[c])
        pltpu.sync_copy(keys_v, keys_hbm.at[pl.ds(c * SLOTS, SLOTS)])
        pltpu.sync_copy(m_v, counts_hbm.at[pl.ds(c * 16, 16)])


@functools.partial(
    pl.kernel,
    mesh=_mesh,
    out_type=jax.ShapeDtypeStruct((OUT_ROWS, 16), jnp.float32),
    scratch_types=[
        pltpu.VMEM((NCLS * SLOTS,), jnp.float32),
        pltpu.VMEM((NCLS * 16,), jnp.int32),
        pltpu.VMEM((PER_W2 + 16, 16), jnp.float32),
        pltpu.VMEM((6, 128), jnp.int32),
        pltpu.VMEM((6, 128), jnp.int32),
        pltpu.VMEM((1024, 16), jnp.float32),
        pltpu.SemaphoreType.DMA,
    ],
    compiler_params=_cparams,
)
def _wbf_topk(rows_hbm, keys_hbm, counts_hbm, out_hbm, keys_v, counts_v,
              rows_v, idx_v, gidx_v, zbuf, sem):
    wid = _wid()
    c = wid // 4
    qt = wid % 4
    iota = lax.iota(jnp.int32, 16)
    pltpu.sync_copy(keys_hbm, keys_v)
    pltpu.sync_copy(counts_hbm, counts_v)
    for t in range(PER_W2 // 16):
        vals = (qt + 4 * t) * 16 + iota
        gidx_v[t // 8, pl.ds((t % 8) * 16, 16)] = vals
    gidx_v[5, pl.ds(112, 16)] = jnp.zeros((16,), jnp.int32)
    gcopies = []
    for j in range(6):
        gcopies.append(
            pltpu.async_copy(rows_hbm.at[c].at[gidx_v.at[j]],
                             rows_v.at[pl.ds(j * 128, 128)], sem))
    for cp in gcopies:
        cp.wait()
    lanes = iota
    dummy = jnp.int32(1000) + wid

    def _count_greater(thr):
        thrv = jnp.full((16,), thr)

        def cg_cls(cc, acc):
            mcc = counts_v[pl.ds(cc * 16, 16)][0]
            nch = (mcc + 15) // 16
            base = cc * SLOTS


            def cg_ch(t, acc):
                v = keys_v[pl.ds(base + t * 16, 16)]
                msk = (v > thrv) & ((t * 16 + iota) < mcc)
                return acc + plsc.all_reduce_population_count(msk)

            return lax.fori_loop(0, nch, cg_ch, acc)

        acc = lax.fori_loop(0, NCLS, cg_cls, jnp.zeros((16,), jnp.int32))
        return jnp.max(acc)

    def bis(it, lohi):
        lo, hi = lohi
        mid = 0.5 * (lo + hi)
        sel = _count_greater(mid) >= POST
        return (jnp.where(sel, mid, lo), jnp.where(sel, hi, mid))

    lo_thr, _ = lax.fori_loop(0, 18, bis,
                              (jnp.float32(-1.0), jnp.float32(1.0)))

    def _rank(key, g):
        keyv = jnp.full((16,), key)
        gv = jnp.full((16,), g)

        def cls_loop(cc, acc):
            mcc = counts_v[pl.ds(cc * 16, 16)][0]
            nch = (mcc + 15) // 16
            base = cc * SLOTS

            def ch(t, acc):
                v = keys_v[pl.ds(base + t * 16, 16)]
                lidx = base + t * 16 + iota
                lvalid = (t * 16 + iota) < mcc
                msk = ((v > keyv) | ((v == keyv) & (lidx < gv))) & lvalid
                return acc + plsc.all_reduce_population_count(msk)

            return lax.fori_loop(0, nch, ch, acc)

        acc = lax.fori_loop(0, NCLS, cls_loop, jnp.zeros((16,), jnp.int32))
        rank = jnp.max(acc)
        return jnp.where(rank < POST, rank, dummy)

    def group(t, carry):
        def one(k, dv):
            g = c * SLOTS + (qt + 4 * t) * 16 + k
            key = plsc.load_gather(keys_v, [jnp.full((16,), g)])[0]
            dest = lax.cond(key > lo_thr, lambda: _rank(key, g),
                            lambda: dummy)
            return jnp.where(lanes == k, dest, dv)

        dv = lax.fori_loop(0, 16, one, jnp.full((16,), dummy))
        idx_v[t // 8, pl.ds((t % 8) * 16, 16)] = dv
        return carry

    lax.fori_loop(0, PER_W2 // 16, group, 0)
    idx_v[5, pl.ds(112, 16)] = jnp.full((16,), dummy)

    copies = []
    for j in range(6):
        copies.append(
            pltpu.async_copy(rows_v.at[pl.ds(j * 128, 128)],
                             out_hbm.at[idx_v.at[j]], sem))
    for cp in copies:
        cp.wait()

    @pl.when(wid == 0)
    def _():
        def zf(t, carry):
            zbuf[t, :] = jnp.zeros((16,), jnp.float32)
            return carry

        lax.fori_loop(0, 1024, zf, 0)
        mtot = jnp.int32(0)
        for cc in range(NCLS):
            mtot = mtot + counts_v[pl.ds(cc * 16, 16)][0]
        zstart = jnp.minimum(mtot, jnp.int32(POST))
        pltpu.sync_copy(zbuf, out_hbm.at[pl.ds(zstart, 1024)])


def kernel(x):
    x3k = jnp.reshape(x[:PRE].astype(jnp.float32), (-1,))
    rows, keys, counts = _wbf_cluster(x3k)
    out = _wbf_topk(rows, keys, counts)
    return out[:POST, :6]


# speedup vs baseline: 501.1821x; 1.0755x over previous
import functools

import jax
import jax.numpy as jnp
from jax import lax
from jax.experimental import pallas as pl
from jax.experimental.pallas import tpu as pltpu
from jax.experimental.pallas import tpu_sc as plsc

PRE = 3000
NCLS = 8
SLOTS = 3008
NCHUNK = SLOTS // 16
POST = 1000
THRESH = 0.5
OUT_ROWS = 2080
PER_W2 = 752
BIG = SLOTS

_mesh = plsc.VectorSubcoreMesh(core_axis_name="c", subcore_axis_name="s")
_cparams = pltpu.CompilerParams(needs_layout_passes=False,
                                use_tc_tiling_on_sc=False)


def _wid():
    return lax.axis_index("s") * 2 + lax.axis_index("c")


@functools.partial(
    pl.kernel,
    mesh=_mesh,
    out_type=[
        jax.ShapeDtypeStruct((NCLS, SLOTS, 16), jnp.float32),
        jax.ShapeDtypeStruct((NCLS * SLOTS,), jnp.float32),
        jax.ShapeDtypeStruct((NCLS * 16,), jnp.int32),
    ],
    scratch_types=[
        pltpu.VMEM((PRE * 6,), jnp.float32),
        pltpu.VMEM((SLOTS, 16), jnp.float32),
        pltpu.VMEM((SLOTS,), jnp.float32),
        pltpu.VMEM((16,), jnp.int32),
        pltpu.VMEM((SLOTS,), jnp.int32),
        pltpu.SMEM((2,), jnp.int32),
    ],
    compiler_params=_cparams,
)
def _wbf_cluster(x_hbm, rows_hbm, keys_hbm, counts_hbm, x_v, state_v,
                 keys_v, m_v, act_v, m_smem):
    wid = _wid()

    @pl.when(wid < NCLS)
    def _():
        c = wid
        cf = c.astype(jnp.float32)
        pltpu.sync_copy(x_hbm, x_v)
        m_smem[0] = jnp.int32(0)
        m_smem[1] = jnp.int32(0)
        iota = lax.iota(jnp.int32, 16)
        lanes = iota
        zeros16i = jnp.zeros((16,), jnp.int32)

        def compact(t, carry):
            rows16 = t * 16 + iota
            lblv = plsc.load_gather(
                x_v, [jnp.minimum(rows16 * 6 + 5, PRE * 6 - 1)])
            msk = (lblv == cf) & (rows16 < PRE)
            cur = m_smem[1]
            pos = cur + plsc.cumsum(msk.astype(jnp.int32)) - 1
            plsc.store_scatter(act_v, [pos], rows16, mask=msk)
            m_smem[1] = cur + jnp.max(
                plsc.all_reduce_population_count(msk))
            return carry

        lax.fori_loop(0, NCHUNK, compact, 0)

        def step(i, carry):
            j = plsc.load_gather(act_v, [jnp.full((16,), i)])[0]
            base = j * 6
            xrow = plsc.load_gather(x_v, [base + jnp.minimum(iota, 5)])

            if True:
                bx1 = xrow[0]
                by1 = xrow[1]
                bx2 = xrow[2]
                by2 = xrow[3]
                sc = xrow[4]
                barea = (bx2 - bx1) * (by2 - by1)
                m = m_smem[0]
                nch = (m + 15) // 16
                vbx1 = jnp.full((16,), bx1)
                vby1 = jnp.full((16,), by1)
                vbx2 = jnp.full((16,), bx2)
                vby2 = jnp.full((16,), by2)
                vba = jnp.full((16,), barea)
                vthr = jnp.float32(THRESH)

                def chunk(t, minvec):
                    ridx = t * 16 + iota
                    mx1 = plsc.load_gather(state_v, [ridx, zeros16i])
                    my1 = plsc.load_gather(state_v, [ridx, zeros16i + 1])
                    mx2 = plsc.load_gather(state_v, [ridx, zeros16i + 2])
                    my2 = plsc.load_gather(state_v, [ridx, zeros16i + 3])
                    ltx = jnp.maximum(mx1, vbx1)
                    lty = jnp.maximum(my1, vby1)
                    rbx = jnp.minimum(mx2, vbx2)
                    rby = jnp.minimum(my2, vby2)
                    iw = jnp.maximum(rbx - ltx, 0.0)
                    ih = jnp.maximum(rby - lty, 0.0)
                    inter = iw * ih
                    mab = (mx2 - mx1) * (my2 - my1)
                    union = vba + mab - inter
                    hits = (inter > vthr * union) & (ridx < m)
                    return jnp.minimum(minvec, jnp.where(hits, ridx, BIG))

                minvec = lax.fori_loop(0, nch, chunk, jnp.full((16,), BIG))
                hitpos = jnp.min(minvec)
                hit = hitpos < m
                idx = jnp.where(hit, hitpos, m)

                oldrow = state_v[idx, :]
                w1 = oldrow[6] + sc * bx1
                w2 = oldrow[7] + sc * by1
                w3 = oldrow[8] + sc * bx2
                w4 = oldrow[9] + sc * by2
                sn = oldrow[10] + sc
                cn = oldrow[11] + 1.0
                numer = jnp.where(
                    lanes == 0, w1,
                    jnp.where(lanes == 1, w2,
                              jnp.where(lanes == 2, w3,
                                        jnp.where(lanes == 3, w4, sn))))
                denom = jnp.where(lanes < 4, sn,
                                  jnp.where(lanes == 4, cn, 1.0))
                q = numer / denom
                hitrow = jnp.where(
                    lanes <= 4, q,
                    jnp.where(lanes == 5, cf,
                              jnp.where(lanes == 6, w1,
                                        jnp.where(lanes == 7, w2,
                                                  jnp.where(lanes == 8, w3,
                                                            jnp.where(lanes == 9, w4,
                                                                      jnp.where(lanes == 10, sn, cn)))))))
                crow = jnp.where(
                    lanes == 0, bx1,
                    jnp.where(lanes == 1, by1,
                              jnp.where(lanes == 2, bx2,
                                        jnp.where(lanes == 3, by2,
                                                  jnp.where(lanes == 4, sc,
                                                            jnp.where(lanes == 5, cf,
                                                                      jnp.where(lanes == 6, sc * bx1,
                                                                                jnp.where(lanes == 7, sc * by1,
                                                                                          jnp.where(lanes == 8, sc * bx2,
                                                                                                    jnp.where(lanes == 9, sc * by2,
                                                                                                              jnp.where(lanes == 10, sc, 1.0)))))))))))
                state_v[idx, :] = jnp.where(hit, hitrow, crow)
                m_smem[0] = jnp.where(hit, m, m + 1)

            return carry

        lax.fori_loop(0, m_smem[1], step, 0)

        m = m_smem[0]

        def key_chunk(t, carry):
            ridx = t * 16 + iota
            scores = plsc.load_gather(state_v, [ridx, zeros16i + 4])
            keys_v[pl.ds(t * 16, 16)] = jnp.where(ridx < m, scores, -1.0)
            return carry

        lax.fori_loop(0, NCHUNK, key_chunk, 0)

        m_v[...] = jnp.full((16,), m, jnp.int32)
        pltpu.sync_copy(state_v, rows_hbm.at[c])
        pltpu.sync_copy(keys_v, keys_hbm.at[pl.ds(c * SLOTS, SLOTS)])
        pltpu.sync_copy(m_v, counts_hbm.at[pl.ds(c * 16, 16)])


@functools.partial(
    pl.kernel,
    mesh=_mesh,
    out_type=jax.ShapeDtypeStruct((OUT_ROWS, 16), jnp.float32),
    scratch_types=[
        pltpu.VMEM((NCLS * SLOTS,), jnp.float32),
        pltpu.VMEM((NCLS * 16,), jnp.int32),
        pltpu.VMEM((PER_W2 + 16, 16), jnp.float32),
        pltpu.VMEM((6, 128), jnp.int32),
        pltpu.VMEM((6, 128), jnp.int32),
        pltpu.VMEM((1024, 16), jnp.float32),
        pltpu.SemaphoreType.DMA,
    ],
    compiler_params=_cparams,
)
def _wbf_topk(rows_hbm, keys_hbm, counts_hbm, out_hbm, keys_v, counts_v,
              rows_v, idx_v, gidx_v, zbuf, sem):
    wid = _wid()
    c = wid // 4
    qt = wid % 4
    iota = lax.iota(jnp.int32, 16)
    pltpu.sync_copy(keys_hbm, keys_v)
    pltpu.sync_copy(counts_hbm, counts_v)
    for t in range(PER_W2 // 16):
        vals = (qt + 4 * t) * 16 + iota
        gidx_v[t // 8, pl.ds((t % 8) * 16, 16)] = vals
    gidx_v[5, pl.ds(112, 16)] = jnp.zeros((16,), jnp.int32)
    gcopies = []
    for j in range(6):
        gcopies.append(
            pltpu.async_copy(rows_hbm.at[c].at[gidx_v.at[j]],
                             rows_v.at[pl.ds(j * 128, 128)], sem))
    for cp in gcopies:
        cp.wait()
    lanes = iota
    dummy = jnp.int32(1000) + wid

    def _count_greater(thr):
        thrv = jnp.full((16,), thr)

        def cg_cls(cc, acc):
            mcc = counts_v[pl.ds(cc * 16, 16)][0]
            nch = (mcc + 15) // 16
            base = cc * SLOTS


            def cg_ch(t, acc):
                v = keys_v[pl.ds(base + t * 16, 16)]
                msk = (v > thrv) & ((t * 16 + iota) < mcc)
                return acc + plsc.all_reduce_population_count(msk)

            return lax.fori_loop(0, nch, cg_ch, acc)

        acc = lax.fori_loop(0, NCLS, cg_cls, jnp.zeros((16,), jnp.int32))
        return jnp.max(acc)

    def bis(it, lohi):
        lo, hi = lohi
        mid = 0.5 * (lo + hi)
        sel = _count_greater(mid) >= POST
        return (jnp.where(sel, mid, lo), jnp.where(sel, hi, mid))

    lo_thr, _ = lax.fori_loop(0, 18, bis,
                              (jnp.float32(-1.0), jnp.float32(1.0)))

    def _rank(key, g):
        keyv = jnp.full((16,), key)
        gv = jnp.full((16,), g)

        def cls_loop(cc, acc):
            mcc = counts_v[pl.ds(cc * 16, 16)][0]
            nch = (mcc + 15) // 16
            base = cc * SLOTS

            def ch(t, acc):
                v = keys_v[pl.ds(base + t * 16, 16)]
                lidx = base + t * 16 + iota
                lvalid = (t * 16 + iota) < mcc
                msk = ((v > keyv) | ((v == keyv) & (lidx < gv))) & lvalid
                return acc + plsc.all_reduce_population_count(msk)

            return lax.fori_loop(0, nch, ch, acc)

        acc = lax.fori_loop(0, NCLS, cls_loop, jnp.zeros((16,), jnp.int32))
        rank = jnp.max(acc)
        return jnp.where(rank < POST, rank, dummy)

    def group(t, carry):
        base_slot = c * SLOTS + (qt + 4 * t) * 16
        kv = keys_v[pl.ds(base_slot, 16)]
        has = jnp.any(kv > lo_thr)

        @pl.when(has)
        def _():
            def one(k, dv):
                g = base_slot + k
                key = plsc.load_gather(keys_v, [jnp.full((16,), g)])[0]
                dest = lax.cond(key > lo_thr, lambda: _rank(key, g),
                                lambda: dummy)
                return jnp.where(lanes == k, dest, dv)

            dv = lax.fori_loop(0, 16, one, jnp.full((16,), dummy))
            idx_v[t // 8, pl.ds((t % 8) * 16, 16)] = dv

        @pl.when(~has)
        def _():
            idx_v[t // 8, pl.ds((t % 8) * 16, 16)] = jnp.full((16,), dummy)

        return carry

    lax.fori_loop(0, PER_W2 // 16, group, 0)
    idx_v[5, pl.ds(112, 16)] = jnp.full((16,), dummy)

    copies = []
    for j in range(6):
        copies.append(
            pltpu.async_copy(rows_v.at[pl.ds(j * 128, 128)],
                             out_hbm.at[idx_v.at[j]], sem))
    for cp in copies:
        cp.wait()

    @pl.when(wid == 0)
    def _():
        mtot = jnp.int32(0)
        for cc in range(NCLS):
            mtot = mtot + counts_v[pl.ds(cc * 16, 16)][0]

        @pl.when(mtot < POST)
        def _():
            def zf(t, carry):
                zbuf[t, :] = jnp.zeros((16,), jnp.float32)
                return carry

            lax.fori_loop(0, 1024, zf, 0)
            pltpu.sync_copy(zbuf, out_hbm.at[pl.ds(mtot, 1024)])


def kernel(x):
    x3k = jnp.reshape(x[:PRE].astype(jnp.float32), (-1,))
    rows, keys, counts = _wbf_cluster(x3k)
    out = _wbf_topk(rows, keys, counts)
    return out[:POST, :6]
